# baseline (speedup 1.0000x reference)
import functools

import jax
import jax.numpy as jnp
from jax import lax
from jax.experimental import pallas as pl
from jax.experimental.pallas import tpu as pltpu
from jax.experimental.pallas import tpu_sc as plsc

N = 10000
NP = 10240
E = 320000
D = 128
NC = 2
NS = 16
NW = NC * NS
CHUNK = E // NW
BLK = 80
NBLK = CHUNK // BLK
ROWS_PER_TILE = NP // NS

_mesh = plsc.VectorSubcoreMesh(core_axis_name="c", subcore_axis_name="s")


@functools.partial(
    pl.kernel,
    mesh=_mesh,
    compiler_params=pltpu.CompilerParams(needs_layout_passes=False),
    out_type=jax.ShapeDtypeStruct((NC, 2, NP), jnp.float32),
    scratch_types=[
        pltpu.VMEM((CHUNK,), jnp.int32),
        pltpu.VMEM((CHUNK,), jnp.float32),
        pltpu.VMEM((ROWS_PER_TILE,), jnp.float32),
        pltpu.VMEM_SHARED((NP,), jnp.float32),
        pltpu.VMEM_SHARED((NP,), jnp.float32),
    ],
)
def _sc_degrees(edges_hbm, out_hbm, idx_v, ones_v, z_v, cnt_out, cnt_in):
    cid = lax.axis_index("c")
    sid = lax.axis_index("s")
    wid = sid * NC + cid

    def fill(i, _):
        ones_v[pl.ds(i * 16, 16)] = jnp.ones((16,), jnp.float32)
        return 0
    lax.fori_loop(0, CHUNK // 16, fill, 0)

    def fillz(i, _):
        z_v[pl.ds(i * 16, 16)] = jnp.zeros((16,), jnp.float32)
        return 0
    lax.fori_loop(0, ROWS_PER_TILE // 16, fillz, 0)

    base = sid * ROWS_PER_TILE
    pltpu.sync_copy(z_v, cnt_out.at[pl.ds(base, ROWS_PER_TILE)])
    pltpu.sync_copy(z_v, cnt_in.at[pl.ds(base, ROWS_PER_TILE)])
    plsc.subcore_barrier()

    pltpu.sync_copy(edges_hbm.at[0, wid], idx_v)
    pltpu.sync_copy(ones_v, cnt_out.at[idx_v], add=True)
    pltpu.sync_copy(edges_hbm.at[1, wid], idx_v)
    pltpu.sync_copy(ones_v, cnt_in.at[idx_v], add=True)
    plsc.subcore_barrier()

    pltpu.sync_copy(cnt_out.at[pl.ds(base, ROWS_PER_TILE)],
                    out_hbm.at[cid, 0, pl.ds(base, ROWS_PER_TILE)])
    pltpu.sync_copy(cnt_in.at[pl.ds(base, ROWS_PER_TILE)],
                    out_hbm.at[cid, 1, pl.ds(base, ROWS_PER_TILE)])


_NB = 512
_NG = NP // _NB


def _tc_dense_body(feat_ref, w2_ref, ww1_ref, ww2_ref, va_ref, wd_ref,
                   dt_ref, degt_ref, h2_ref, p_ref, t_ref):
    h2 = jnp.dot(feat_ref[...], w2_ref[...], preferred_element_type=jnp.float32)
    h2_ref[...] = h2
    va = va_ref[...]
    v1 = jnp.dot(ww1_ref[0:D, :], va, preferred_element_type=jnp.float32)
    v2 = jnp.dot(ww2_ref[0:D, :], va, preferred_element_type=jnp.float32)
    v3 = jnp.dot(ww2_ref[D:2 * D, :], va, preferred_element_type=jnp.float32)
    v4 = jnp.concatenate([v1, v2, v3, jnp.zeros((D, 1), jnp.float32)], axis=1)
    p = jnp.dot(h2, v4, preferred_element_type=jnp.float32)
    out_deg = degt_ref[:, 0:1] + degt_ref[:, 2:3]
    d2 = lax.rsqrt(jnp.maximum(out_deg, 1.0))
    sel3 = (lax.broadcasted_iota(jnp.int32, (1, 4), 1) == 3).astype(jnp.float32)
    p_ref[...] = p + d2 * sel3
    vb = jnp.dot(ww1_ref[D:2 * D, :], va, preferred_element_type=jnp.float32)
    wv = jnp.dot(wd_ref[...], vb, preferred_element_type=jnp.float32)
    tcol = jnp.dot(dt_ref[...], wv, preferred_element_type=jnp.float32)
    t_ref[...] = jnp.concatenate([tcol, jnp.zeros((7, 1), jnp.float32)], axis=0)


def _tc_dense(feat_p, W2, Ww1, Ww2, va, Wd, dist_table, degT):
    return pl.pallas_call(
        _tc_dense_body,
        grid=(_NG,),
        in_specs=[
            pl.BlockSpec((_NB, D), lambda i: (i, 0)),
            pl.BlockSpec((D, D), lambda i: (0, 0)),
            pl.BlockSpec((2 * D, D), lambda i: (0, 0)),
            pl.BlockSpec((2 * D, D), lambda i: (0, 0)),
            pl.BlockSpec((D, 1), lambda i: (0, 0)),
            pl.BlockSpec((16, D), lambda i: (0, 0)),
            pl.BlockSpec((33, 16), lambda i: (0, 0)),
            pl.BlockSpec((_NB, 4), lambda i: (i, 0)),
        ],
        out_specs=[
            pl.BlockSpec((_NB, D), lambda i: (i, 0)),
            pl.BlockSpec((_NB, 4), lambda i: (i, 0)),
            pl.BlockSpec((40, 1), lambda i: (0, 0)),
        ],
        out_shape=[
            jax.ShapeDtypeStruct((NP, D), jnp.float32),
            jax.ShapeDtypeStruct((NP, 4), jnp.float32),
            jax.ShapeDtypeStruct((40, 1), jnp.float32),
        ],
    )(feat_p, W2, Ww1, Ww2, va, Wd, dist_table, degT)


@functools.partial(
    pl.kernel,
    mesh=_mesh,
    compiler_params=pltpu.CompilerParams(needs_layout_passes=False),
    out_type=jax.ShapeDtypeStruct((NW, NBLK, 2, BLK), jnp.float32),
    scratch_types=[
        pltpu.VMEM((NP * 4,), jnp.float32),
        pltpu.VMEM((N * 3,), jnp.float32),
        pltpu.VMEM((40,), jnp.float32),
        pltpu.VMEM((32,), jnp.float32),
        pltpu.VMEM((NBLK, BLK), jnp.int32),
        pltpu.VMEM((NBLK, BLK), jnp.int32),
        pltpu.VMEM((NBLK, BLK), jnp.int32),
        pltpu.VMEM((2, BLK), jnp.float32),
    ],
)
def _sc_coefs(edges_hbm, mids_hbm, p_hbm, t_hbm, bnd_hbm, loc_hbm,
              out_hbm, p_v, loc_v, t_v, bnd_v, src_v, dst_v, mid_v, c_v):
    cid = lax.axis_index("c")
    sid = lax.axis_index("s")
    wid = sid * NC + cid

    pltpu.sync_copy(p_hbm, p_v)
    pltpu.sync_copy(loc_hbm, loc_v)
    pltpu.sync_copy(t_hbm, t_v)
    pltpu.sync_copy(bnd_hbm, bnd_v)
    pltpu.sync_copy(edges_hbm.at[0, wid], src_v)
    pltpu.sync_copy(edges_hbm.at[1, wid], dst_v)
    pltpu.sync_copy(mids_hbm.at[wid], mid_v)

    def block(blk, _):
        b_lo = bnd_v[pl.ds(0, 16)]
        b_hi = bnd_v[pl.ds(16, 16)]
        for v in range(BLK // 16):
            sl = pl.ds(16 * v, 16)
            si = src_v[blk, sl]
            di = dst_v[blk, sl]
            mi = mid_v[blk, sl]
            s3i = si * 3
            d3i = di * 3
            dx = (plsc.load_gather(loc_v, [d3i])
                  - plsc.load_gather(loc_v, [s3i]))
            dy = (plsc.load_gather(loc_v, [d3i + 1])
                  - plsc.load_gather(loc_v, [s3i + 1]))
            dz = (plsc.load_gather(loc_v, [d3i + 2])
                  - plsc.load_gather(loc_v, [s3i + 2]))
            dist2 = dx * dx + dy * dy + dz * dz
            bucket = jnp.zeros((16,), jnp.int32)
            for j in range(32):
                bj = b_lo[j] if j < 16 else b_hi[j - 16]
                bucket = bucket + jnp.where(dist2 > bj * bj, 1, 0).astype(jnp.int32)
            tval = plsc.load_gather(t_v, [bucket])
            s4i = si * 4
            s1 = plsc.load_gather(p_v, [di * 4])
            s2 = plsc.load_gather(p_v, [mi * 4 + 1])
            s3 = plsc.load_gather(p_v, [s4i + 2])
            d2s = plsc.load_gather(p_v, [s4i + 3])
            score = s1 + s2 + s3 + tval
            beta = 1.0 / (1.0 + jnp.exp(-score))
            c_v[0, sl] = d2s * beta
            c_v[1, sl] = d2s
        pltpu.sync_copy(c_v, out_hbm.at[wid, blk])
        return 0

    lax.fori_loop(0, NBLK, block, 0)


BLK2 = 64
GRP = 10
NGRP = 16
CHUNK2 = BLK2 * GRP * NGRP


@functools.partial(
    pl.kernel,
    mesh=_mesh,
    compiler_params=pltpu.CompilerParams(needs_layout_passes=False),
    out_type=jax.ShapeDtypeStruct((NC, NP, D), jnp.float32),
    scratch_types=[
        pltpu.VMEM((GRP, 2, 128), jnp.int32),
        pltpu.VMEM((GRP, 128), jnp.float32),
        pltpu.VMEM((BLK2,), jnp.int32),
        pltpu.VMEM((BLK2,), jnp.int32),
        pltpu.VMEM((BLK2, D), jnp.float32),
        pltpu.VMEM((BLK2, D), jnp.float32),
        pltpu.VMEM((BLK2, D), jnp.float32),
        pltpu.VMEM((BLK2, D), jnp.float32),
        pltpu.SemaphoreType.DMA,
        pltpu.SemaphoreType.DMA,
        pltpu.SemaphoreType.DMA,
        pltpu.SemaphoreType.DMA,
        pltpu.VMEM_SHARED((NP, D), jnp.float32),
    ],
)
def _sc_edges(idx_hbm, coef_hbm, h2_hbm, out_hbm,
              islab, cslab, dstb0, dstb1, h2s0, h2s1, h2m0, h2m1,
              gs0, gs1, gm0, gm1, acc_sh):
    cid = lax.axis_index("c")
    sid = lax.axis_index("s")
    wid = sid * NC + cid
    dstb = (dstb0, dstb1)
    h2s = (h2s0, h2s1)
    h2m = (h2m0, h2m1)
    gs = (gs0, gs1)
    gm = (gm0, gm1)

    def fillz(i, _):
        for j in range(D // 16):
            h2m0[i, pl.ds(16 * j, 16)] = jnp.zeros((16,), jnp.float32)
        return 0
    lax.fori_loop(0, BLK2, fillz, 0)
    base = sid * ROWS_PER_TILE
    for k in range(ROWS_PER_TILE // BLK2):
        pltpu.sync_copy(h2m0, acc_sh.at[pl.ds(base + k * BLK2, BLK2)])
    plsc.subcore_barrier()

    def fire(j, p):
        for k in range(BLK2 // 16):
            dstb[p][pl.ds(16 * k, 16)] = islab[j, 0, pl.ds(64 + 16 * k, 16)]
        a = pltpu.async_copy(h2_hbm.at[islab.at[j, 0, pl.ds(0, BLK2)]],
                             h2s[p], gs[p])
        b = pltpu.async_copy(h2_hbm.at[islab.at[j, 1, pl.ds(0, BLK2)]],
                             h2m[p], gm[p])
        return a, b

    def grp_body(g, _):
        pltpu.sync_copy(idx_hbm.at[wid, g], islab)
        pltpu.sync_copy(coef_hbm.at[wid, g], cslab)
        pend = fire(0, 0)
        for j in range(GRP):
            p = j % 2
            cur = pend
            if j + 1 < GRP:
                pend = fire(j + 1, 1 - p)
            cur[0].wait()
            cur[1].wait()

            def vgrp(v, _):
                c1v = cslab[j, pl.ds(16 * v, 16)]
                c2v = cslab[j, pl.ds(64 + 16 * v, 16)]
                for e16 in range(16):
                    e = 16 * v + e16
                    c1 = c1v[e16]
                    c2 = c2v[e16]
                    for q in range(D // 16):
                        fs = pl.ds(16 * q, 16)
                        h2m[p][e, fs] = c1 * h2s[p][e, fs] + c2 * h2m[p][e, fs]
                return 0
            lax.fori_loop(0, BLK2 // 16, vgrp, 0)
            pltpu.sync_copy(h2m[p], acc_sh.at[dstb[p]], add=True)
        return 0

    lax.fori_loop(0, NGRP, grp_body, 0)
    plsc.subcore_barrier()

    def wr(k, _):
        r = base + k * 8
        pltpu.sync_copy(acc_sh.at[pl.ds(r, 8)],
                        out_hbm.at[cid, pl.ds(r, 8)])
        return 0
    lax.fori_loop(0, ROWS_PER_TILE // 8, wr, 0)


def _tc_final_body(acc_ref, degt_ref, out_ref):
    a = acc_ref[0] + acc_ref[1]
    in_deg = degt_ref[:, 1:2] + degt_ref[:, 3:4]
    out_ref[...] = lax.rsqrt(jnp.maximum(in_deg, 1.0)) * a


def _tc_final(accp, degT):
    return pl.pallas_call(
        _tc_final_body,
        grid=(_NG,),
        in_specs=[
            pl.BlockSpec((NC, _NB, D), lambda i: (0, i, 0)),
            pl.BlockSpec((_NB, 4), lambda i: (i, 0)),
        ],
        out_specs=pl.BlockSpec((_NB, D), lambda i: (i, 0)),
        out_shape=jax.ShapeDtypeStruct((NP, D), jnp.float32),
    )(accp, degT)


def kernel(feat, loc, edge_index, mid_ids, boundaries, dist_table,
           W2, Wd, Ww1, Ww2, va):
    edges2 = edge_index.reshape(2, NW, NBLK, BLK)
    edges_flat = edge_index.reshape(2, NW, CHUNK)
    mids2 = mid_ids.reshape(NW, NBLK, BLK)

    degp = _sc_degrees(edges_flat)
    degT = jnp.transpose(degp.reshape(2 * NC, NP))

    feat_p = jnp.pad(feat, ((0, NP - N), (0, 0)))
    h2, P, tpad = _tc_dense(feat_p, W2, Ww1, Ww2, va, Wd, dist_table, degT)

    coef = _sc_coefs(edges2, mids2, P.reshape(NP * 4),
                     tpad.reshape(40), boundaries,
                     loc.reshape(N * 3))
    ef = edge_index.reshape(2, NW, CHUNK)
    mf = mid_ids.reshape(NW, CHUNK)
    pad3 = ((0, 0), (0, 0), (0, CHUNK2 - CHUNK))
    srcp = jnp.pad(ef[0:1], pad3, constant_values=N)[0].reshape(NW, GRP * NGRP, BLK2)
    dstp = jnp.pad(ef[1:2], pad3, constant_values=N)[0].reshape(NW, GRP * NGRP, BLK2)
    midp = jnp.pad(mf, ((0, 0), (0, CHUNK2 - CHUNK)),
                   constant_values=N).reshape(NW, GRP * NGRP, BLK2)
    row0 = jnp.concatenate([srcp, dstp], axis=-1)
    row1 = jnp.concatenate([midp, midp], axis=-1)
    idxpack = jnp.stack([row0, row1], axis=2).reshape(NW, NGRP, GRP, 2, 128)
    c1f = jnp.pad(coef[:, :, 0, :].reshape(NW, CHUNK),
                  ((0, 0), (0, CHUNK2 - CHUNK))).reshape(NW, GRP * NGRP, BLK2)
    c2f = jnp.pad(coef[:, :, 1, :].reshape(NW, CHUNK),
                  ((0, 0), (0, CHUNK2 - CHUNK))).reshape(NW, GRP * NGRP, BLK2)
    cpack = jnp.concatenate([c1f, c2f], axis=-1).reshape(NW, NGRP, GRP, 128)
    accp = _sc_edges(idxpack, cpack, h2)

    out = _tc_final(accp, degT)
    return out[:N]


# speedup vs baseline: 1.7048x; 1.7048x over previous
import functools

import numpy as np
import jax
import jax.numpy as jnp
from jax import lax
from jax.experimental import pallas as pl
from jax.experimental.pallas import tpu as pltpu
from jax.experimental.pallas import tpu_sc as plsc

N = 10000
NP = 10240
E = 320000
D = 128
NC = 2
NS = 16
NW = NC * NS
CHUNK = E // NW
BLK = 80
NBLK = CHUNK // BLK
ROWS_PER_TILE = NP // NS

_mesh = plsc.VectorSubcoreMesh(core_axis_name="c", subcore_axis_name="s")


@functools.partial(
    pl.kernel,
    mesh=_mesh,
    compiler_params=pltpu.CompilerParams(needs_layout_passes=False),
    out_type=jax.ShapeDtypeStruct((NC, 2, NP), jnp.float32),
    scratch_types=[
        pltpu.VMEM((CHUNK,), jnp.int32),
        pltpu.VMEM((CHUNK,), jnp.float32),
        pltpu.VMEM((ROWS_PER_TILE,), jnp.float32),
        pltpu.VMEM_SHARED((NP,), jnp.float32),
        pltpu.VMEM_SHARED((NP,), jnp.float32),
    ],
)
def _sc_degrees(edges_hbm, out_hbm, idx_v, ones_v, z_v, cnt_out, cnt_in):
    cid = lax.axis_index("c")
    sid = lax.axis_index("s")
    wid = sid * NC + cid

    def fill(i, _):
        ones_v[pl.ds(i * 16, 16)] = jnp.ones((16,), jnp.float32)
        return 0
    lax.fori_loop(0, CHUNK // 16, fill, 0)

    def fillz(i, _):
        z_v[pl.ds(i * 16, 16)] = jnp.zeros((16,), jnp.float32)
        return 0
    lax.fori_loop(0, ROWS_PER_TILE // 16, fillz, 0)

    base = sid * ROWS_PER_TILE
    pltpu.sync_copy(z_v, cnt_out.at[pl.ds(base, ROWS_PER_TILE)])
    pltpu.sync_copy(z_v, cnt_in.at[pl.ds(base, ROWS_PER_TILE)])
    plsc.subcore_barrier()

    pltpu.sync_copy(edges_hbm.at[0, wid], idx_v)
    pltpu.sync_copy(ones_v, cnt_out.at[idx_v], add=True)
    pltpu.sync_copy(edges_hbm.at[1, wid], idx_v)
    pltpu.sync_copy(ones_v, cnt_in.at[idx_v], add=True)
    plsc.subcore_barrier()

    pltpu.sync_copy(cnt_out.at[pl.ds(base, ROWS_PER_TILE)],
                    out_hbm.at[cid, 0, pl.ds(base, ROWS_PER_TILE)])
    pltpu.sync_copy(cnt_in.at[pl.ds(base, ROWS_PER_TILE)],
                    out_hbm.at[cid, 1, pl.ds(base, ROWS_PER_TILE)])


_NB = 512
_NG = NP // _NB


def _tc_dense_body(feat_ref, w2_ref, ww1_ref, ww2_ref, va_ref, wd_ref,
                   dt_ref, degt_ref, h2_ref, p_ref, t_ref):
    h2 = jnp.dot(feat_ref[...], w2_ref[...], preferred_element_type=jnp.float32)
    h2_ref[...] = h2
    va = va_ref[...]
    v1 = jnp.dot(ww1_ref[0:D, :], va, preferred_element_type=jnp.float32)
    v2 = jnp.dot(ww2_ref[0:D, :], va, preferred_element_type=jnp.float32)
    v3 = jnp.dot(ww2_ref[D:2 * D, :], va, preferred_element_type=jnp.float32)
    v4 = jnp.concatenate([v1, v2, v3, jnp.zeros((D, 1), jnp.float32)], axis=1)
    p = jnp.dot(h2, v4, preferred_element_type=jnp.float32)
    out_deg = degt_ref[:, 0:1] + degt_ref[:, 2:3]
    d2 = lax.rsqrt(jnp.maximum(out_deg, 1.0))
    sel3 = (lax.broadcasted_iota(jnp.int32, (1, 4), 1) == 3).astype(jnp.float32)
    p_ref[...] = p + d2 * sel3
    vb = jnp.dot(ww1_ref[D:2 * D, :], va, preferred_element_type=jnp.float32)
    wv = jnp.dot(wd_ref[...], vb, preferred_element_type=jnp.float32)
    tcol = jnp.dot(dt_ref[...], wv, preferred_element_type=jnp.float32)
    t_ref[...] = jnp.concatenate([tcol, jnp.zeros((7, 1), jnp.float32)], axis=0)


def _tc_dense(feat_p, W2, Ww1, Ww2, va, Wd, dist_table, degT):
    return pl.pallas_call(
        _tc_dense_body,
        grid=(_NG,),
        in_specs=[
            pl.BlockSpec((_NB, D), lambda i: (i, 0)),
            pl.BlockSpec((D, D), lambda i: (0, 0)),
            pl.BlockSpec((2 * D, D), lambda i: (0, 0)),
            pl.BlockSpec((2 * D, D), lambda i: (0, 0)),
            pl.BlockSpec((D, 1), lambda i: (0, 0)),
            pl.BlockSpec((16, D), lambda i: (0, 0)),
            pl.BlockSpec((33, 16), lambda i: (0, 0)),
            pl.BlockSpec((_NB, 4), lambda i: (i, 0)),
        ],
        out_specs=[
            pl.BlockSpec((_NB, D), lambda i: (i, 0)),
            pl.BlockSpec((_NB, 4), lambda i: (i, 0)),
            pl.BlockSpec((40, 1), lambda i: (0, 0)),
        ],
        out_shape=[
            jax.ShapeDtypeStruct((NP, D), jnp.float32),
            jax.ShapeDtypeStruct((NP, 4), jnp.float32),
            jax.ShapeDtypeStruct((40, 1), jnp.float32),
        ],
    )(feat_p, W2, Ww1, Ww2, va, Wd, dist_table, degT)


@functools.partial(
    pl.kernel,
    mesh=_mesh,
    compiler_params=pltpu.CompilerParams(needs_layout_passes=False),
    out_type=jax.ShapeDtypeStruct((NW, NBLK, 2, BLK), jnp.float32),
    scratch_types=[
        pltpu.VMEM((NP * 4,), jnp.float32),
        pltpu.VMEM((N * 3,), jnp.float32),
        pltpu.VMEM((40,), jnp.float32),
        pltpu.VMEM((32,), jnp.float32),
        pltpu.VMEM((NBLK, BLK), jnp.int32),
        pltpu.VMEM((NBLK, BLK), jnp.int32),
        pltpu.VMEM((NBLK, BLK), jnp.int32),
        pltpu.VMEM((2, BLK), jnp.float32),
    ],
)
def _sc_coefs(edges_hbm, mids_hbm, p_hbm, t_hbm, bnd_hbm, loc_hbm,
              out_hbm, p_v, loc_v, t_v, bnd_v, src_v, dst_v, mid_v, c_v):
    cid = lax.axis_index("c")
    sid = lax.axis_index("s")
    wid = sid * NC + cid

    pltpu.sync_copy(p_hbm, p_v)
    pltpu.sync_copy(loc_hbm, loc_v)
    pltpu.sync_copy(t_hbm, t_v)
    pltpu.sync_copy(bnd_hbm, bnd_v)
    pltpu.sync_copy(edges_hbm.at[0, wid], src_v)
    pltpu.sync_copy(edges_hbm.at[1, wid], dst_v)
    pltpu.sync_copy(mids_hbm.at[wid], mid_v)

    def block(blk, _):
        b_lo = bnd_v[pl.ds(0, 16)]
        b_hi = bnd_v[pl.ds(16, 16)]
        for v in range(BLK // 16):
            sl = pl.ds(16 * v, 16)
            si = src_v[blk, sl]
            di = dst_v[blk, sl]
            mi = mid_v[blk, sl]
            s3i = si * 3
            d3i = di * 3
            dx = (plsc.load_gather(loc_v, [d3i])
                  - plsc.load_gather(loc_v, [s3i]))
            dy = (plsc.load_gather(loc_v, [d3i + 1])
                  - plsc.load_gather(loc_v, [s3i + 1]))
            dz = (plsc.load_gather(loc_v, [d3i + 2])
                  - plsc.load_gather(loc_v, [s3i + 2]))
            dist2 = dx * dx + dy * dy + dz * dz
            bucket = jnp.zeros((16,), jnp.int32)
            for j in range(32):
                bj = b_lo[j] if j < 16 else b_hi[j - 16]
                bucket = bucket + jnp.where(dist2 > bj * bj, 1, 0).astype(jnp.int32)
            tval = plsc.load_gather(t_v, [bucket])
            s4i = si * 4
            s1 = plsc.load_gather(p_v, [di * 4])
            s2 = plsc.load_gather(p_v, [mi * 4 + 1])
            s3 = plsc.load_gather(p_v, [s4i + 2])
            d2s = plsc.load_gather(p_v, [s4i + 3])
            score = s1 + s2 + s3 + tval
            beta = 1.0 / (1.0 + jnp.exp(-score))
            c_v[0, sl] = d2s * beta
            c_v[1, sl] = d2s
        pltpu.sync_copy(c_v, out_hbm.at[wid, blk])
        return 0

    lax.fori_loop(0, NBLK, block, 0)


SB = 5
NSB = NBLK // SB


@functools.partial(
    pl.kernel,
    mesh=_mesh,
    compiler_params=pltpu.CompilerParams(needs_layout_passes=False),
    out_type=jax.ShapeDtypeStruct((NC, NP, D), jnp.float32),
    scratch_types=[
        pltpu.VMEM((SB, BLK), jnp.int32),
        pltpu.VMEM((SB, BLK), jnp.int32),
        pltpu.VMEM((SB, BLK), jnp.int32),
        pltpu.VMEM((SB, 2, BLK), jnp.float32),
        pltpu.VMEM((BLK, D), jnp.float32),
        pltpu.VMEM((BLK, D), jnp.float32),
        pltpu.VMEM_SHARED((NP, D), jnp.float32),
        pltpu.SemaphoreType.DMA,
        pltpu.SemaphoreType.DMA,
    ],
)
def _sc_edges(edges_hbm, mids_hbm, coef_hbm, h2_hbm, out_hbm,
              src_v, dst_v, mid_v, c_v, h2s_v, h2m_v, acc_sh,
              sem1, sem2):
    cid = lax.axis_index("c")
    sid = lax.axis_index("s")
    wid = sid * NC + cid

    def fillz(i, _):
        for j in range(D // 16):
            h2m_v[i, pl.ds(16 * j, 16)] = jnp.zeros((16,), jnp.float32)
        return 0
    lax.fori_loop(0, BLK, fillz, 0)
    base = sid * ROWS_PER_TILE
    for k in range(ROWS_PER_TILE // BLK):
        pltpu.sync_copy(h2m_v, acc_sh.at[pl.ds(base + k * BLK, BLK)])
    plsc.subcore_barrier()

    def slab(sb, _):
        pltpu.sync_copy(edges_hbm.at[0, wid, sb], src_v)
        pltpu.sync_copy(edges_hbm.at[1, wid, sb], dst_v)
        pltpu.sync_copy(mids_hbm.at[wid, sb], mid_v)
        pltpu.sync_copy(coef_hbm.at[wid, sb], c_v)

        def block(b, _):
            g1 = pltpu.async_copy(h2_hbm.at[src_v.at[b]], h2s_v, sem1)
            g2 = pltpu.async_copy(h2_hbm.at[mid_v.at[b]], h2m_v, sem2)
            g1.wait()
            g2.wait()

            def vgrp(v, _):
                c1v = c_v[b, 0, pl.ds(16 * v, 16)]
                c2v = c_v[b, 1, pl.ds(16 * v, 16)]
                for e16 in range(16):
                    e = 16 * v + e16
                    c1 = c1v[e16]
                    c2 = c2v[e16]
                    for q in range(D // 16):
                        fs = pl.ds(16 * q, 16)
                        h2m_v[e, fs] = c1 * h2s_v[e, fs] + c2 * h2m_v[e, fs]
                return 0
            lax.fori_loop(0, BLK // 16, vgrp, 0)

            pltpu.sync_copy(h2m_v, acc_sh.at[dst_v.at[b]], add=True)
            return 0

        lax.fori_loop(0, SB, block, 0)
        return 0

    lax.fori_loop(0, NSB, slab, 0)
    plsc.subcore_barrier()

    def wr(k, _):
        r = base + k * 8
        pltpu.sync_copy(acc_sh.at[pl.ds(r, 8)],
                        out_hbm.at[cid, pl.ds(r, 8)])
        return 0
    lax.fori_loop(0, ROWS_PER_TILE // 8, wr, 0)


def _tc_final_body(acc_ref, degt_ref, out_ref):
    a = acc_ref[0] + acc_ref[1]
    in_deg = degt_ref[:, 1:2] + degt_ref[:, 3:4]
    out_ref[...] = lax.rsqrt(jnp.maximum(in_deg, 1.0)) * a


def _tc_final(accp, degT):
    return pl.pallas_call(
        _tc_final_body,
        grid=(_NG,),
        in_specs=[
            pl.BlockSpec((NC, _NB, D), lambda i: (0, i, 0)),
            pl.BlockSpec((_NB, 4), lambda i: (i, 0)),
        ],
        out_specs=pl.BlockSpec((_NB, D), lambda i: (i, 0)),
        out_shape=jax.ShapeDtypeStruct((NP, D), jnp.float32),
    )(accp, degT)


def kernel(feat, loc, edge_index, mid_ids, boundaries, dist_table,
           W2, Wd, Ww1, Ww2, va):
    edges2 = edge_index.reshape(2, NW, NBLK, BLK)
    edges_flat = edge_index.reshape(2, NW, CHUNK)
    mids2 = mid_ids.reshape(NW, NBLK, BLK)

    degp = _sc_degrees(edges_flat)
    degT = jnp.transpose(degp.reshape(2 * NC, NP))

    feat_p = jnp.pad(feat, ((0, NP - N), (0, 0)))
    h2, P, tpad = _tc_dense(feat_p, W2, Ww1, Ww2, va, Wd, dist_table, degT)

    coef = _sc_coefs(edges2, mids2, P.reshape(NP * 4),
                     tpad.reshape(40), boundaries,
                     loc.reshape(N * 3))
    edges5 = edge_index.reshape(2, NW, NSB, SB, BLK)
    mids5 = mid_ids.reshape(NW, NSB, SB, BLK)
    coef5 = coef.reshape(NW, NSB, SB, 2, BLK)
    accp = _sc_edges(edges5, mids5, coef5, h2)


    out = _tc_final(accp, degT)
    return out[:N]


# speedup vs baseline: 1.7852x; 1.0472x over previous
import functools

import numpy as np
import jax
import jax.numpy as jnp
from jax import lax
from jax.experimental import pallas as pl
from jax.experimental.pallas import tpu as pltpu
from jax.experimental.pallas import tpu_sc as plsc

N = 10000
NP = 10240
E = 320000
D = 128
NC = 2
NS = 16
NW = NC * NS
CHUNK = E // NW
BLK = 80
NBLK = CHUNK // BLK
ROWS_PER_TILE = NP // NS

_mesh = plsc.VectorSubcoreMesh(core_axis_name="c", subcore_axis_name="s")


@functools.partial(
    pl.kernel,
    mesh=_mesh,
    compiler_params=pltpu.CompilerParams(needs_layout_passes=False),
    out_type=jax.ShapeDtypeStruct((NC, 2, NP), jnp.float32),
    scratch_types=[
        pltpu.VMEM((CHUNK,), jnp.int32),
        pltpu.VMEM((CHUNK,), jnp.float32),
        pltpu.VMEM((ROWS_PER_TILE,), jnp.float32),
        pltpu.VMEM_SHARED((NP,), jnp.float32),
        pltpu.VMEM_SHARED((NP,), jnp.float32),
    ],
)
def _sc_degrees(edges_hbm, out_hbm, idx_v, ones_v, z_v, cnt_out, cnt_in):
    cid = lax.axis_index("c")
    sid = lax.axis_index("s")
    wid = sid * NC + cid

    def fill(i, _):
        ones_v[pl.ds(i * 16, 16)] = jnp.ones((16,), jnp.float32)
        return 0
    lax.fori_loop(0, CHUNK // 16, fill, 0)

    def fillz(i, _):
        z_v[pl.ds(i * 16, 16)] = jnp.zeros((16,), jnp.float32)
        return 0
    lax.fori_loop(0, ROWS_PER_TILE // 16, fillz, 0)

    base = sid * ROWS_PER_TILE
    pltpu.sync_copy(z_v, cnt_out.at[pl.ds(base, ROWS_PER_TILE)])
    pltpu.sync_copy(z_v, cnt_in.at[pl.ds(base, ROWS_PER_TILE)])
    plsc.subcore_barrier()

    pltpu.sync_copy(edges_hbm.at[0, wid], idx_v)
    pltpu.sync_copy(ones_v, cnt_out.at[idx_v], add=True)
    pltpu.sync_copy(edges_hbm.at[1, wid], idx_v)
    pltpu.sync_copy(ones_v, cnt_in.at[idx_v], add=True)
    plsc.subcore_barrier()

    pltpu.sync_copy(cnt_out.at[pl.ds(base, ROWS_PER_TILE)],
                    out_hbm.at[cid, 0, pl.ds(base, ROWS_PER_TILE)])
    pltpu.sync_copy(cnt_in.at[pl.ds(base, ROWS_PER_TILE)],
                    out_hbm.at[cid, 1, pl.ds(base, ROWS_PER_TILE)])


_NB = 512
_NG = NP // _NB


def _tc_dense_body(feat_ref, w2_ref, ww1_ref, ww2_ref, va_ref, wd_ref,
                   dt_ref, degt_ref, h2_ref, p_ref, t_ref):
    h2 = jnp.dot(feat_ref[...], w2_ref[...], preferred_element_type=jnp.float32)
    h2_ref[...] = h2
    va = va_ref[...]
    v1 = jnp.dot(ww1_ref[0:D, :], va, preferred_element_type=jnp.float32)
    v2 = jnp.dot(ww2_ref[0:D, :], va, preferred_element_type=jnp.float32)
    v3 = jnp.dot(ww2_ref[D:2 * D, :], va, preferred_element_type=jnp.float32)
    v4 = jnp.concatenate([v1, v2, v3, jnp.zeros((D, 1), jnp.float32)], axis=1)
    p = jnp.dot(h2, v4, preferred_element_type=jnp.float32)
    out_deg = degt_ref[:, 0:1] + degt_ref[:, 2:3]
    d2 = lax.rsqrt(jnp.maximum(out_deg, 1.0))
    sel3 = (lax.broadcasted_iota(jnp.int32, (1, 4), 1) == 3).astype(jnp.float32)
    p_ref[...] = p + d2 * sel3
    vb = jnp.dot(ww1_ref[D:2 * D, :], va, preferred_element_type=jnp.float32)
    wv = jnp.dot(wd_ref[...], vb, preferred_element_type=jnp.float32)
    tcol = jnp.dot(dt_ref[...], wv, preferred_element_type=jnp.float32)
    t_ref[...] = jnp.concatenate([tcol, jnp.zeros((7, 1), jnp.float32)], axis=0)


def _tc_dense(feat_p, W2, Ww1, Ww2, va, Wd, dist_table, degT):
    return pl.pallas_call(
        _tc_dense_body,
        grid=(_NG,),
        in_specs=[
            pl.BlockSpec((_NB, D), lambda i: (i, 0)),
            pl.BlockSpec((D, D), lambda i: (0, 0)),
            pl.BlockSpec((2 * D, D), lambda i: (0, 0)),
            pl.BlockSpec((2 * D, D), lambda i: (0, 0)),
            pl.BlockSpec((D, 1), lambda i: (0, 0)),
            pl.BlockSpec((16, D), lambda i: (0, 0)),
            pl.BlockSpec((33, 16), lambda i: (0, 0)),
            pl.BlockSpec((_NB, 4), lambda i: (i, 0)),
        ],
        out_specs=[
            pl.BlockSpec((_NB, D), lambda i: (i, 0)),
            pl.BlockSpec((_NB, 4), lambda i: (i, 0)),
            pl.BlockSpec((40, 1), lambda i: (0, 0)),
        ],
        out_shape=[
            jax.ShapeDtypeStruct((NP, D), jnp.float32),
            jax.ShapeDtypeStruct((NP, 4), jnp.float32),
            jax.ShapeDtypeStruct((40, 1), jnp.float32),
        ],
    )(feat_p, W2, Ww1, Ww2, va, Wd, dist_table, degT)


@functools.partial(
    pl.kernel,
    mesh=_mesh,
    compiler_params=pltpu.CompilerParams(needs_layout_passes=False),
    out_type=jax.ShapeDtypeStruct((NW, NBLK, 2, BLK), jnp.float32),
    scratch_types=[
        pltpu.VMEM((NP * 4,), jnp.float32),
        pltpu.VMEM((N * 3,), jnp.float32),
        pltpu.VMEM((40,), jnp.float32),
        pltpu.VMEM((32,), jnp.float32),
        pltpu.VMEM((NBLK, BLK), jnp.int32),
        pltpu.VMEM((NBLK, BLK), jnp.int32),
        pltpu.VMEM((NBLK, BLK), jnp.int32),
        pltpu.VMEM((2, BLK), jnp.float32),
    ],
)
def _sc_coefs(edges_hbm, mids_hbm, p_hbm, t_hbm, bnd_hbm, loc_hbm,
              out_hbm, p_v, loc_v, t_v, bnd_v, src_v, dst_v, mid_v, c_v):
    cid = lax.axis_index("c")
    sid = lax.axis_index("s")
    wid = sid * NC + cid

    pltpu.sync_copy(p_hbm, p_v)
    pltpu.sync_copy(loc_hbm, loc_v)
    pltpu.sync_copy(t_hbm, t_v)
    pltpu.sync_copy(bnd_hbm, bnd_v)
    pltpu.sync_copy(edges_hbm.at[0, wid], src_v)
    pltpu.sync_copy(edges_hbm.at[1, wid], dst_v)
    pltpu.sync_copy(mids_hbm.at[wid], mid_v)

    def block(blk, _):
        b_lo = bnd_v[pl.ds(0, 16)]
        b_hi = bnd_v[pl.ds(16, 16)]
        for v in range(BLK // 16):
            sl = pl.ds(16 * v, 16)
            si = src_v[blk, sl]
            di = dst_v[blk, sl]
            mi = mid_v[blk, sl]
            s3i = si * 3
            d3i = di * 3
            dx = (plsc.load_gather(loc_v, [d3i])
                  - plsc.load_gather(loc_v, [s3i]))
            dy = (plsc.load_gather(loc_v, [d3i + 1])
                  - plsc.load_gather(loc_v, [s3i + 1]))
            dz = (plsc.load_gather(loc_v, [d3i + 2])
                  - plsc.load_gather(loc_v, [s3i + 2]))
            dist2 = dx * dx + dy * dy + dz * dz
            bucket = jnp.zeros((16,), jnp.int32)
            for j in range(32):
                bj = b_lo[j] if j < 16 else b_hi[j - 16]
                bucket = bucket + jnp.where(dist2 > bj * bj, 1, 0).astype(jnp.int32)
            tval = plsc.load_gather(t_v, [bucket])
            s4i = si * 4
            s1 = plsc.load_gather(p_v, [di * 4])
            s2 = plsc.load_gather(p_v, [mi * 4 + 1])
            s3 = plsc.load_gather(p_v, [s4i + 2])
            d2s = plsc.load_gather(p_v, [s4i + 3])
            score = s1 + s2 + s3 + tval
            beta = 1.0 / (1.0 + jnp.exp(-score))
            c_v[0, sl] = d2s * beta
            c_v[1, sl] = d2s
        pltpu.sync_copy(c_v, out_hbm.at[wid, blk])
        return 0

    lax.fori_loop(0, NBLK, block, 0)


SB = 5
NSB = NBLK // SB


@functools.partial(
    pl.kernel,
    mesh=_mesh,
    compiler_params=pltpu.CompilerParams(needs_layout_passes=False),
    out_type=jax.ShapeDtypeStruct((NC, NP, D), jnp.float32),
    scratch_types=[
        pltpu.VMEM((SB, BLK), jnp.int32),
        pltpu.VMEM((SB, BLK), jnp.int32),
        pltpu.VMEM((SB, BLK), jnp.int32),
        pltpu.VMEM((SB, 2, BLK), jnp.float32),
        pltpu.VMEM((BLK, D), jnp.float32),
        pltpu.VMEM((BLK, D), jnp.float32),
        pltpu.VMEM((BLK, D), jnp.float32),
        pltpu.VMEM_SHARED((NP, D), jnp.float32),
        pltpu.SemaphoreType.DMA,
        pltpu.SemaphoreType.DMA,
        pltpu.SemaphoreType.DMA,
    ],
)
def _sc_edges(edges_hbm, mids_hbm, coef_hbm, h2_hbm, out_hbm,
              src_v, dst_v, mid_v, c_v, h2s_v, h2m_v, he_v, acc_sh,
              sem1, sem2, sem3):
    cid = lax.axis_index("c")
    sid = lax.axis_index("s")
    wid = sid * NC + cid

    def fillz(i, _):
        for j in range(D // 16):
            he_v[i, pl.ds(16 * j, 16)] = jnp.zeros((16,), jnp.float32)
        return 0
    lax.fori_loop(0, BLK, fillz, 0)
    base = sid * ROWS_PER_TILE
    for k in range(ROWS_PER_TILE // BLK):
        pltpu.sync_copy(he_v, acc_sh.at[pl.ds(base + k * BLK, BLK)])
    plsc.subcore_barrier()

    def compute(b):
        def vgrp(v, _):
            c1v = c_v[b, 0, pl.ds(16 * v, 16)]
            c2v = c_v[b, 1, pl.ds(16 * v, 16)]
            for e16 in range(16):
                e = 16 * v + e16
                c1 = c1v[e16]
                c2 = c2v[e16]
                for q in range(D // 16):
                    fs = pl.ds(16 * q, 16)
                    he_v[e, fs] = c1 * h2s_v[e, fs] + c2 * h2m_v[e, fs]
            return 0
        lax.fori_loop(0, BLK // 16, vgrp, 0)

    def drain_scatter():
        pltpu.make_async_copy(he_v, acc_sh.at[pl.ds(0, BLK)], sem3).wait()

    def slab(sb, _):
        pltpu.sync_copy(edges_hbm.at[0, wid, sb], src_v)
        pltpu.sync_copy(edges_hbm.at[1, wid, sb], dst_v)
        pltpu.sync_copy(mids_hbm.at[wid, sb], mid_v)
        pltpu.sync_copy(coef_hbm.at[wid, sb], c_v)

        g1 = pltpu.async_copy(h2_hbm.at[src_v.at[0]], h2s_v, sem1)
        g2 = pltpu.async_copy(h2_hbm.at[mid_v.at[0]], h2m_v, sem2)
        g1.wait()
        g2.wait()
        compute(0)
        pltpu.async_copy(he_v, acc_sh.at[dst_v.at[0]], sem3, add=True)

        def block(b, _):
            g1 = pltpu.async_copy(h2_hbm.at[src_v.at[b]], h2s_v, sem1)
            g2 = pltpu.async_copy(h2_hbm.at[mid_v.at[b]], h2m_v, sem2)
            g1.wait()
            g2.wait()
            drain_scatter()
            compute(b)
            pltpu.async_copy(he_v, acc_sh.at[dst_v.at[b]], sem3, add=True)
            return 0

        lax.fori_loop(1, SB, block, 0)
        drain_scatter()
        return 0

    lax.fori_loop(0, NSB, slab, 0)
    plsc.subcore_barrier()

    def wr(k, _):
        r = base + k * 8
        pltpu.sync_copy(acc_sh.at[pl.ds(r, 8)],
                        out_hbm.at[cid, pl.ds(r, 8)])
        return 0
    lax.fori_loop(0, ROWS_PER_TILE // 8, wr, 0)


def _tc_final_body(acc_ref, degt_ref, out_ref):
    a = acc_ref[0] + acc_ref[1]
    in_deg = degt_ref[:, 1:2] + degt_ref[:, 3:4]
    out_ref[...] = lax.rsqrt(jnp.maximum(in_deg, 1.0)) * a


def _tc_final(accp, degT):
    return pl.pallas_call(
        _tc_final_body,
        grid=(_NG,),
        in_specs=[
            pl.BlockSpec((NC, _NB, D), lambda i: (0, i, 0)),
            pl.BlockSpec((_NB, 4), lambda i: (i, 0)),
        ],
        out_specs=pl.BlockSpec((_NB, D), lambda i: (i, 0)),
        out_shape=jax.ShapeDtypeStruct((NP, D), jnp.float32),
    )(accp, degT)


def kernel(feat, loc, edge_index, mid_ids, boundaries, dist_table,
           W2, Wd, Ww1, Ww2, va):
    edges2 = edge_index.reshape(2, NW, NBLK, BLK)
    edges_flat = edge_index.reshape(2, NW, CHUNK)
    mids2 = mid_ids.reshape(NW, NBLK, BLK)

    degp = _sc_degrees(edges_flat)
    degT = jnp.transpose(degp.reshape(2 * NC, NP))

    feat_p = jnp.pad(feat, ((0, NP - N), (0, 0)))
    h2, P, tpad = _tc_dense(feat_p, W2, Ww1, Ww2, va, Wd, dist_table, degT)

    coef = _sc_coefs(edges2, mids2, P.reshape(NP * 4),
                     tpad.reshape(40), boundaries,
                     loc.reshape(N * 3))
    edges5 = edge_index.reshape(2, NW, NSB, SB, BLK)
    mids5 = mid_ids.reshape(NW, NSB, SB, BLK)
    coef5 = coef.reshape(NW, NSB, SB, 2, BLK)
    accp = _sc_edges(edges5, mids5, coef5, h2)


    out = _tc_final(accp, degT)
    return out[:N]
